# baseline (speedup 1.0000x reference)
import functools

import jax
import jax.numpy as jnp
from jax import lax
from jax.experimental import pallas as pl
from jax.experimental.pallas import tpu as pltpu
from jax.experimental.pallas import tpu_sc as plsc

E = 128
S0, S1, SO = 4, 3, 4
B = 50000
NC, NS = 2, 16
NW = NC * NS
C = 24
NBUF = 3
G = -(-B // C)
T = NBUF * (-(-G // (NW * NBUF)))

OUT_PATHS = (
    ((0, 0, 1.0), (3, 1, 0.4)),
    ((1, 0, 0.5), (0, 1, 0.2)),
    ((2, 1, -0.3), (1, 2, 1.1)),
    ((3, 2, 0.7), (2, 0, -0.9)),
)

_mesh = plsc.VectorSubcoreMesh(core_axis_name="c", subcore_axis_name="s")


@functools.partial(
    pl.kernel,
    mesh=_mesh,
    out_type=jax.ShapeDtypeStruct((B, SO * E), jnp.float32),
    scratch_types=[
        pltpu.VMEM((NBUF, C, S0 * E), jnp.float32),
        pltpu.VMEM((NBUF, C, S1 * E), jnp.float32),
        pltpu.VMEM((NBUF, C, SO * E), jnp.float32),
        pltpu.SemaphoreType.DMA((NBUF,)),
        pltpu.SemaphoreType.DMA((NBUF,)),
        pltpu.SemaphoreType.DMA((NBUF,)),
    ],
)
def _sc_poly(in0_hbm, in1_hbm, out_hbm, x0_v, x1_v, o_v, s0, s1, so):
    wid = lax.axis_index("s") * NC + lax.axis_index("c")

    def g_of(t):
        return wid + t * NW

    def issue_in(t, b):
        @pl.when(g_of(t) < G)
        def _():
            st = jnp.minimum(g_of(t) * C, B - C)
            pltpu.make_async_copy(
                in0_hbm.at[pl.ds(st, C)], x0_v.at[b], s0.at[b]).start()
            pltpu.make_async_copy(
                in1_hbm.at[pl.ds(st, C)], x1_v.at[b], s1.at[b]).start()

    def wait_in(t, b):
        @pl.when(g_of(t) < G)
        def _():
            pltpu.make_async_copy(
                in0_hbm.at[pl.ds(0, C)], x0_v.at[b], s0.at[b]).wait()
            pltpu.make_async_copy(
                in1_hbm.at[pl.ds(0, C)], x1_v.at[b], s1.at[b]).wait()

    def issue_out(t, b):
        @pl.when(g_of(t) < G)
        def _():
            st = jnp.minimum(g_of(t) * C, B - C)
            pltpu.make_async_copy(
                o_v.at[b], out_hbm.at[pl.ds(st, C)], so.at[b]).start()

    def wait_out(t, b):
        @pl.when((t >= 0) & (g_of(t) < G))
        def _():
            pltpu.make_async_copy(
                o_v.at[b], out_hbm.at[pl.ds(0, C)], so.at[b]).wait()

    def compute(t, b):
        @pl.when(g_of(t) < G)
        def _():
            @plsc.parallel_loop(0, C * (E // 16), unroll=8)
            def rw_body(rv):
                r = rv // (E // 16)
                o = (rv % (E // 16)) * 16
                a = [x0_v[b, r, pl.ds(i * E + o, 16)] for i in range(S0)]
                c = [x1_v[b, r, pl.ds(j * E + o, 16)] for j in range(S1)]
                for k, ((i1, j1, c1), (i2, j2, c2)) in enumerate(OUT_PATHS):
                    acc = jnp.float32(c1) * (a[i1] * c[j1]) \
                        + jnp.float32(c2) * (a[i2] * c[j2])
                    o_v[b, r, pl.ds(k * E + o, 16)] = acc

    for b in range(NBUF):
        issue_in(b, b)

    def pipe_body(tt, carry):
        for b in range(NBUF):
            t = NBUF * tt + b
            wait_in(t, b)
            wait_out(t - NBUF, b)
            compute(t, b)
            issue_out(t, b)
            issue_in(t + NBUF, b)
        return carry

    lax.fori_loop(0, T // NBUF, pipe_body, 0)
    for b in range(NBUF):
        wait_out(T - NBUF + b, b)


def kernel(in0, in1):
    return _sc_poly(in0, in1)


# speedup vs baseline: 1.0154x; 1.0154x over previous
import functools

import jax
import jax.numpy as jnp
from jax import lax
from jax.experimental import pallas as pl
from jax.experimental.pallas import tpu as pltpu
from jax.experimental.pallas import tpu_sc as plsc

E = 128
S0, S1, SO = 4, 3, 4
B = 50000
NC, NS = 2, 16
NW = NC * NS
C = 24
NBUF = 3
G = -(-B // C)
T = NBUF * (-(-G // (NW * NBUF)))

OUT_PATHS = (
    ((0, 0, 1.0), (3, 1, 0.4)),
    ((1, 0, 0.5), (0, 1, 0.2)),
    ((2, 1, -0.3), (1, 2, 1.1)),
    ((3, 2, 0.7), (2, 0, -0.9)),
)

_mesh = plsc.VectorSubcoreMesh(core_axis_name="c", subcore_axis_name="s")


@functools.partial(
    pl.kernel,
    mesh=_mesh,
    out_type=jax.ShapeDtypeStruct((B, SO * E), jnp.float32),
    scratch_types=[
        pltpu.VMEM((NBUF, C, S0 * E), jnp.float32),
        pltpu.VMEM((NBUF, C, S1 * E), jnp.float32),
        pltpu.VMEM((NBUF, C, SO * E), jnp.float32),
        pltpu.SemaphoreType.DMA((NBUF,)),
        pltpu.SemaphoreType.DMA((NBUF,)),
        pltpu.SemaphoreType.DMA((NBUF,)),
    ],
)
def _sc_poly(in0_hbm, in1_hbm, out_hbm, x0_v, x1_v, o_v, s0, s1, so):
    wid = lax.axis_index("s") * NC + lax.axis_index("c")

    def g_of(t):
        return wid + t * NW

    def issue_in(t, b):
        @pl.when(g_of(t) < G)
        def _():
            st = jnp.minimum(g_of(t) * C, B - C)
            pltpu.make_async_copy(
                in0_hbm.at[pl.ds(st, C)], x0_v.at[b], s0.at[b]).start()
            pltpu.make_async_copy(
                in1_hbm.at[pl.ds(st, C)], x1_v.at[b], s1.at[b]).start()

    def wait_in(t, b):
        @pl.when(g_of(t) < G)
        def _():
            pltpu.make_async_copy(
                in0_hbm.at[pl.ds(0, C)], x0_v.at[b], s0.at[b]).wait()
            pltpu.make_async_copy(
                in1_hbm.at[pl.ds(0, C)], x1_v.at[b], s1.at[b]).wait()

    def issue_out(t, b):
        @pl.when(g_of(t) < G)
        def _():
            st = jnp.minimum(g_of(t) * C, B - C)
            pltpu.make_async_copy(
                o_v.at[b], out_hbm.at[pl.ds(st, C)], so.at[b]).start()

    def wait_out(t, b):
        @pl.when((t >= 0) & (g_of(t) < G))
        def _():
            pltpu.make_async_copy(
                o_v.at[b], out_hbm.at[pl.ds(0, C)], so.at[b]).wait()

    def compute(t, b):
        @pl.when(g_of(t) < G)
        def _():
            @plsc.parallel_loop(0, C * (E // 16), unroll=2)
            def rw_body(rv):
                r = rv // (E // 16)
                o = (rv % (E // 16)) * 16
                a = [x0_v[b, r, pl.ds(i * E + o, 16)] for i in range(S0)]
                c = [x1_v[b, r, pl.ds(j * E + o, 16)] for j in range(S1)]
                for k, ((i1, j1, c1), (i2, j2, c2)) in enumerate(OUT_PATHS):
                    acc = jnp.float32(c1) * (a[i1] * c[j1]) \
                        + jnp.float32(c2) * (a[i2] * c[j2])
                    o_v[b, r, pl.ds(k * E + o, 16)] = acc

    for b in range(NBUF):
        issue_in(b, b)

    def pipe_body(tt, carry):
        for b in range(NBUF):
            t = NBUF * tt + b
            wait_in(t, b)
            wait_out(t - NBUF, b)
            compute(t, b)
            issue_out(t, b)
            issue_in(t + NBUF, b)
        return carry

    lax.fori_loop(0, T // NBUF, pipe_body, 0)
    for b in range(NBUF):
        wait_out(T - NBUF + b, b)


def kernel(in0, in1):
    return _sc_poly(in0, in1)
